# baseline (speedup 1.0000x reference)
import functools

import jax
import jax.numpy as jnp
from jax import lax
from jax.experimental import pallas as pl
from jax.experimental.pallas import tpu as pltpu
from jax.experimental.pallas import tpu_sc as plsc

B, S, D = 4, 8192, 1024
NC, NS, L = 2, 16, 16
NW = NC * NS
R = 4
SETS = 4
AHEAD = 2

S_SC = 4096
S_TC = S - S_SC
NT = S_SC // (NW * R)
G = NT // SETS

SB = 512
NSB = S_TC // SB


def _sc_add(x, pos):
    mesh = plsc.VectorSubcoreMesh(core_axis_name="c", subcore_axis_name="s")

    @functools.partial(
        pl.kernel,
        mesh=mesh,
        out_type=jax.ShapeDtypeStruct((B, S, D), jnp.float32),
        scratch_types=(
            [pltpu.VMEM((B, R, D), jnp.float32)] * SETS
            + [pltpu.VMEM((R, D), jnp.float32)] * SETS
            + [pltpu.SemaphoreType.DMA] * (2 * SETS)
        ),
    )
    def k(x_hbm, pos_hbm, out_hbm, *scr):
        xs = scr[:SETS]
        ps = scr[SETS:2 * SETS]
        sin = scr[2 * SETS:3 * SETS]
        sout = scr[3 * SETS:4 * SETS]
        wid = lax.axis_index("s") * NC + lax.axis_index("c")

        def issue_in(j, s):
            r0 = (j * NW + wid) * R
            pltpu.async_copy(pos_hbm.at[pl.ds(r0, R)], ps[s], sin[s])
            pltpu.async_copy(x_hbm.at[:, pl.ds(r0, R)], xs[s], sin[s])

        def wait_in(s):
            pltpu.make_async_copy(pos_hbm.at[pl.ds(0, R)], ps[s], sin[s]).wait()
            pltpu.make_async_copy(x_hbm.at[:, pl.ds(0, R)], xs[s], sin[s]).wait()

        def issue_out(j, s):
            r0 = (j * NW + wid) * R
            pltpu.async_copy(xs[s], out_hbm.at[:, pl.ds(r0, R)], sout[s])

        def wait_out(s):
            pltpu.make_async_copy(x_hbm.at[:, pl.ds(0, R)], xs[s], sout[s]).wait()

        def compute(s):
            xv, pv_ref = xs[s], ps[s]

            @plsc.parallel_loop(0, D // L, step=1)
            def body(j):
                o = j * L
                for r in range(R):
                    pv = pv_ref[r, pl.ds(o, L)]
                    for b in range(B):
                        plsc.addupdate(xv.at[b, r, pl.ds(o, L)], pv)

        for j in range(AHEAD):
            issue_in(j, j % SETS)

        def g_body(g, carry):
            i0 = g * SETS
            for s in range(SETS):
                i = i0 + s
                nxt = i + AHEAD
                t = (s + AHEAD) % SETS

                @pl.when(jnp.logical_and(nxt >= SETS, nxt < NT))
                def _():
                    wait_out(t)

                @pl.when(nxt < NT)
                def _():
                    issue_in(nxt, t)

                wait_in(s)
                compute(s)
                issue_out(i, s)
            return carry

        lax.fori_loop(0, G, g_body, 0, unroll=False)
        for s in range(SETS):
            wait_out(s)

    return k(x, pos)


def _tc_add(x, pos):
    def body(x_ref, p_ref, o_ref):
        o_ref[...] = x_ref[...] + p_ref[...][None, :, :]

    return pl.pallas_call(
        body,
        grid=(NSB, B),
        in_specs=[
            pl.BlockSpec((1, SB, D), lambda i, b: (b, S_SC // SB + i, 0)),
            pl.BlockSpec((SB, D), lambda i, b: (S_SC // SB + i, 0)),
        ],
        out_specs=pl.BlockSpec((1, SB, D), lambda i, b: (b, i, 0)),
        out_shape=jax.ShapeDtypeStruct((B, S_TC, D), jnp.float32),
    )(x, pos)


def kernel(x, pos_embedding):
    tc_out = _tc_add(x, pos_embedding)
    sc_out = _sc_add(x, pos_embedding)
    return lax.dynamic_update_slice(sc_out, tc_out, (0, S_SC, 0))


# speedup vs baseline: 1.3040x; 1.3040x over previous
import functools

import jax
import jax.numpy as jnp
from jax import lax
from jax.experimental import pallas as pl
from jax.experimental.pallas import tpu as pltpu
from jax.experimental.pallas import tpu_sc as plsc

B, S, D = 4, 8192, 1024
NC, NS, L = 2, 16, 16
NW = NC * NS
R = 4
NT = S // (NW * R)
SETS = 4
AHEAD = 2
G = NT // SETS


def _sc_add(x, pos):
    mesh = plsc.VectorSubcoreMesh(core_axis_name="c", subcore_axis_name="s")

    @functools.partial(
        pl.kernel,
        mesh=mesh,
        out_type=jax.ShapeDtypeStruct((B, S, D), jnp.float32),
        scratch_types=(
            [pltpu.VMEM((B, R, D), jnp.float32)] * SETS
            + [pltpu.VMEM((R, D), jnp.float32)] * SETS
            + [pltpu.SemaphoreType.DMA] * (2 * SETS)
        ),
    )
    def k(x_hbm, pos_hbm, out_hbm, *scr):
        xs = scr[:SETS]
        ps = scr[SETS:2 * SETS]
        sin = scr[2 * SETS:3 * SETS]
        sout = scr[3 * SETS:4 * SETS]
        wid = lax.axis_index("s") * NC + lax.axis_index("c")

        def issue_in(j, s):
            r0 = (j * NW + wid) * R
            pltpu.async_copy(pos_hbm.at[pl.ds(r0, R)], ps[s], sin[s])
            pltpu.async_copy(x_hbm.at[:, pl.ds(r0, R)], xs[s], sin[s])

        def wait_in(s):
            pltpu.make_async_copy(pos_hbm.at[pl.ds(0, R)], ps[s], sin[s]).wait()
            pltpu.make_async_copy(x_hbm.at[:, pl.ds(0, R)], xs[s], sin[s]).wait()

        def issue_out(j, s):
            r0 = (j * NW + wid) * R
            pltpu.async_copy(xs[s], out_hbm.at[:, pl.ds(r0, R)], sout[s])

        def wait_out(s):
            pltpu.make_async_copy(x_hbm.at[:, pl.ds(0, R)], xs[s], sout[s]).wait()

        def compute(s):
            xv, pv_ref = xs[s], ps[s]

            @plsc.parallel_loop(0, D // L, step=1)
            def body(j):
                o = j * L
                for r in range(R):
                    pv = pv_ref[r, pl.ds(o, L)]
                    for b in range(B):
                        plsc.addupdate(xv.at[b, r, pl.ds(o, L)], pv)

        for j in range(AHEAD):
            issue_in(j, j % SETS)

        def g_body(g, carry):
            i0 = g * SETS
            for s in range(SETS):
                i = i0 + s
                nxt = i + AHEAD
                t = (s + AHEAD) % SETS

                @pl.when(jnp.logical_and(nxt >= SETS, nxt < NT))
                def _():
                    wait_out(t)

                @pl.when(nxt < NT)
                def _():
                    issue_in(nxt, t)

                wait_in(s)
                compute(s)
                issue_out(i, s)
            return carry

        lax.fori_loop(0, G, g_body, 0, unroll=False)
        for s in range(SETS):
            wait_out(s)

    return k(x, pos)


def kernel(x, pos_embedding):
    return _sc_add(x, pos_embedding)
